# baseline (speedup 1.0000x reference)
import functools

import jax
import jax.numpy as jnp
from jax import lax
from jax.experimental import pallas as pl
from jax.experimental.pallas import tpu as pltpu
from jax.experimental.pallas import tpu_sc as plsc


def _sc_gather(user_id, item_id, user_embed, item_embed):
    B = user_id.shape[0]
    D = user_embed.shape[1]
    info = plsc.get_sparse_core_info()
    NC, NS = info.num_cores, info.num_subcores
    NW = NC * NS
    bw = B // NW

    mesh = plsc.VectorSubcoreMesh(core_axis_name="c", subcore_axis_name="s")

    @functools.partial(
        pl.kernel,
        out_type=(
            jax.ShapeDtypeStruct((B, D), jnp.float32),
            jax.ShapeDtypeStruct((B, D), jnp.float32),
        ),
        mesh=mesh,
        scratch_types=[
            pltpu.VMEM((bw,), jnp.int32),
            pltpu.VMEM((bw,), jnp.int32),
            pltpu.VMEM((bw, D), jnp.float32),
            pltpu.VMEM((bw, D), jnp.float32),
            pltpu.SemaphoreType.DMA,
            pltpu.SemaphoreType.DMA,
        ],
        compiler_params=pltpu.CompilerParams(use_tc_tiling_on_sc=False),
    )
    def gather_k(uid_hbm, iid_hbm, uemb_hbm, iemb_hbm, pu_hbm, qi_hbm,
                 uidx_v, iidx_v, urows_v, irows_v, usem, isem):
        wid = lax.axis_index("s") * NC + lax.axis_index("c")
        base = wid * bw
        pltpu.sync_copy(uid_hbm.at[pl.ds(base, bw)], uidx_v)
        pltpu.sync_copy(iid_hbm.at[pl.ds(base, bw)], iidx_v)
        ucp = pltpu.async_copy(uemb_hbm.at[uidx_v], urows_v, usem)
        icp = pltpu.async_copy(iemb_hbm.at[iidx_v], irows_v, isem)
        ucp.wait()
        icp.wait()
        pltpu.sync_copy(urows_v, pu_hbm.at[pl.ds(base, bw)])
        pltpu.sync_copy(irows_v, qi_hbm.at[pl.ds(base, bw)])

    return gather_k(user_id, item_id, user_embed, item_embed)


def _tc_matmul(pu, qi):
    B, D = pu.shape
    BM = BN = 512
    grid = (B // BM, B // BN)

    def body(p_ref, q_ref, o_ref):
        o_ref[...] = lax.dot_general(
            p_ref[...], q_ref[...],
            dimension_numbers=(((1,), (1,)), ((), ())),
            preferred_element_type=jnp.float32,
        )

    return pl.pallas_call(
        body,
        grid=grid,
        in_specs=[
            pl.BlockSpec((BM, D), lambda i, j: (i, 0)),
            pl.BlockSpec((BN, D), lambda i, j: (j, 0)),
        ],
        out_specs=pl.BlockSpec((BM, BN), lambda i, j: (i, j)),
        out_shape=jax.ShapeDtypeStruct((B, B), jnp.float32),
    )(pu, qi)


def kernel(user_id, item_id, user_embed, item_embed):
    uid = user_id.astype(jnp.int32)
    iid = item_id.astype(jnp.int32)
    pu, qi = _sc_gather(uid, iid, user_embed, item_embed)
    return _tc_matmul(pu, qi)


# speedup vs baseline: 1.0258x; 1.0258x over previous
import functools

import jax
import jax.numpy as jnp
from jax import lax
from jax.experimental import pallas as pl
from jax.experimental.pallas import tpu as pltpu
from jax.experimental.pallas import tpu_sc as plsc

_PACK = 4


def _sc_gather_packed(puid, piid, uemb_packed, iemb_packed):
    B = puid.shape[0]
    W = uemb_packed.shape[1]
    info = plsc.get_sparse_core_info()
    NC, NS = info.num_cores, info.num_subcores
    NW = NC * NS
    bw = B // NW

    mesh = plsc.VectorSubcoreMesh(core_axis_name="c", subcore_axis_name="s")

    @functools.partial(
        pl.kernel,
        out_type=(
            jax.ShapeDtypeStruct((B, W), jnp.float32),
            jax.ShapeDtypeStruct((B, W), jnp.float32),
        ),
        mesh=mesh,
        scratch_types=[
            pltpu.VMEM((bw,), jnp.int32),
            pltpu.VMEM((bw,), jnp.int32),
            pltpu.VMEM((bw, W), jnp.float32),
            pltpu.VMEM((bw, W), jnp.float32),
            pltpu.SemaphoreType.DMA,
            pltpu.SemaphoreType.DMA,
        ],
    )
    def gather_k(uid_hbm, iid_hbm, uemb_hbm, iemb_hbm, pu_hbm, qi_hbm,
                 uidx_v, iidx_v, urows_v, irows_v, usem, isem):
        wid = lax.axis_index("s") * NC + lax.axis_index("c")
        base = wid * bw
        pltpu.sync_copy(uid_hbm.at[pl.ds(base, bw)], uidx_v)
        pltpu.sync_copy(iid_hbm.at[pl.ds(base, bw)], iidx_v)
        ucp = pltpu.async_copy(uemb_hbm.at[uidx_v], urows_v, usem)
        icp = pltpu.async_copy(iemb_hbm.at[iidx_v], irows_v, isem)
        ucp.wait()
        icp.wait()
        pltpu.sync_copy(urows_v, pu_hbm.at[pl.ds(base, bw)])
        pltpu.sync_copy(irows_v, qi_hbm.at[pl.ds(base, bw)])

    return gather_k(puid, piid, uemb_packed, iemb_packed)


def _extract(packed, off, d):
    s0 = packed[:, 0 * d:1 * d]
    s1 = packed[:, 1 * d:2 * d]
    s2 = packed[:, 2 * d:3 * d]
    s3 = packed[:, 3 * d:4 * d]
    return jnp.where(off == 0, s0,
                     jnp.where(off == 1, s1,
                               jnp.where(off == 2, s2, s3)))


def _tc_matmul(pu_packed, qi_packed, off_u, off_i, d):
    B, W = pu_packed.shape
    BM = 512
    grid = (B // BM,)

    def body(p_ref, q_ref, ou_ref, oi_ref, o_ref):
        p = _extract(p_ref[...], ou_ref[...], d)
        q = _extract(q_ref[...], oi_ref[...], d)
        o_ref[...] = lax.dot_general(
            p, q,
            dimension_numbers=(((1,), (1,)), ((), ())),
            preferred_element_type=jnp.float32,
        )

    return pl.pallas_call(
        body,
        grid=grid,
        in_specs=[
            pl.BlockSpec((BM, W), lambda i: (i, 0)),
            pl.BlockSpec((B, W), lambda i: (0, 0)),
            pl.BlockSpec((BM, 1), lambda i: (i, 0)),
            pl.BlockSpec((B, 1), lambda i: (0, 0)),
        ],
        out_specs=pl.BlockSpec((BM, B), lambda i: (i, 0)),
        out_shape=jax.ShapeDtypeStruct((B, B), jnp.float32),
    )(pu_packed, qi_packed, off_u, off_i)


def kernel(user_id, item_id, user_embed, item_embed):
    uid = user_id.astype(jnp.int32)
    iid = item_id.astype(jnp.int32)
    n_u, d = user_embed.shape
    n_i, _ = item_embed.shape
    uemb_packed = user_embed.reshape(n_u // _PACK, _PACK * d)
    iemb_packed = item_embed.reshape(n_i // _PACK, _PACK * d)
    pu_packed, qi_packed = _sc_gather_packed(
        uid // _PACK, iid // _PACK, uemb_packed, iemb_packed)
    off_u = (uid % _PACK).reshape(-1, 1)
    off_i = (iid % _PACK).reshape(-1, 1)
    return _tc_matmul(pu_packed, qi_packed, off_u, off_i, d)


# speedup vs baseline: 8.6915x; 8.4730x over previous
import functools

import jax
import jax.numpy as jnp
from jax import lax
from jax.experimental import pallas as pl
from jax.experimental.pallas import tpu as pltpu
from jax.experimental.pallas import tpu_sc as plsc

_GRP = 16
_LANE = 128


def _sc_gather_cols(uid, iid, uemb_t, iemb_t):
    B = uid.shape[0]
    D = uemb_t.shape[0]
    info = plsc.get_sparse_core_info()
    NC, NS = info.num_cores, info.num_subcores
    NW = NC * NS
    bw = B // NW

    mesh = plsc.VectorSubcoreMesh(core_axis_name="c", subcore_axis_name="s")

    @functools.partial(
        pl.kernel,
        out_type=(
            jax.ShapeDtypeStruct((D, B), jnp.float32),
            jax.ShapeDtypeStruct((D, B), jnp.float32),
        ),
        mesh=mesh,
        scratch_types=[
            pltpu.VMEM((bw,), jnp.int32),
            pltpu.VMEM((bw,), jnp.int32),
            pltpu.VMEM((_GRP, D, _LANE), jnp.float32),
            pltpu.VMEM((D, bw), jnp.float32),
            pltpu.VMEM((D, bw), jnp.float32),
            pltpu.SemaphoreType.DMA,
        ],
        compiler_params=pltpu.CompilerParams(needs_layout_passes=False),
    )
    def gather_k(uid_hbm, iid_hbm, uemb_hbm, iemb_hbm, put_hbm, qit_hbm,
                 uidx_v, iidx_v, ring_v, outu_v, outi_v, sem):
        wid = lax.axis_index("s") * NC + lax.axis_index("c")
        base = pl.multiple_of(wid * bw, bw)
        pltpu.sync_copy(uid_hbm.at[pl.ds(base, bw)], uidx_v)
        pltpu.sync_copy(iid_hbm.at[pl.ds(base, bw)], iidx_v)

        iota = lax.iota(jnp.int32, 16)
        d_lo = iota
        d_hi = iota + 16

        def run_table(tab, idx_v, out_v):
            def group(gi, _):
                idx16 = idx_v[pl.ds(gi * _GRP, _GRP)]
                for k in range(_GRP):
                    cb = pl.multiple_of((idx16[k] >> 7) << 7, _LANE)
                    pltpu.async_copy(tab.at[:, pl.ds(cb, _LANE)],
                                     ring_v.at[k], sem)
                for k in range(_GRP):
                    pltpu.make_async_copy(tab.at[:, pl.ds(0, _LANE)],
                                          ring_v.at[k], sem).wait()
                for k in range(_GRP):
                    o = idx16[k] & (_LANE - 1)
                    slot_v = jnp.full((16,), k, jnp.int32)
                    col_v = jnp.full((16,), o, jnp.int32)
                    g_v = jnp.full((16,), gi * _GRP + k, jnp.int32)
                    lo = plsc.load_gather(ring_v, [slot_v, d_lo, col_v])
                    hi = plsc.load_gather(ring_v, [slot_v, d_hi, col_v])
                    plsc.store_scatter(out_v, [d_lo, g_v], lo)
                    plsc.store_scatter(out_v, [d_hi, g_v], hi)
                return _

            lax.fori_loop(0, bw // _GRP, group, None)

        run_table(uemb_hbm, uidx_v, outu_v)
        run_table(iemb_hbm, iidx_v, outi_v)

        pltpu.sync_copy(outu_v, put_hbm.at[:, pl.ds(base, bw)])
        pltpu.sync_copy(outi_v, qit_hbm.at[:, pl.ds(base, bw)])

    return gather_k(uid, iid, uemb_t, iemb_t)


def _tc_matmul(pu_t, qi_t):
    D, B = pu_t.shape
    BM = 512
    grid = (B // BM,)

    def body(p_ref, q_ref, o_ref):
        o_ref[...] = lax.dot_general(
            p_ref[...], q_ref[...],
            dimension_numbers=(((0,), (0,)), ((), ())),
            preferred_element_type=jnp.float32,
        )

    return pl.pallas_call(
        body,
        grid=grid,
        in_specs=[
            pl.BlockSpec((D, BM), lambda i: (0, i)),
            pl.BlockSpec((D, B), lambda i: (0, 0)),
        ],
        out_specs=pl.BlockSpec((BM, B), lambda i: (i, 0)),
        out_shape=jax.ShapeDtypeStruct((B, B), jnp.float32),
    )(pu_t, qi_t)


def kernel(user_id, item_id, user_embed, item_embed):
    uid = user_id.astype(jnp.int32)
    iid = item_id.astype(jnp.int32)
    pu_t, qi_t = _sc_gather_cols(uid, iid, user_embed.T, item_embed.T)
    return _tc_matmul(pu_t, qi_t)
